# baseline (speedup 1.0000x reference)
import jax
import jax.numpy as jnp
from jax.experimental import pallas as pl
from jax.experimental.pallas import tpu as pltpu

NSTREAMS = 2
BLOCK_ROWS = 200


def _make_kernel(nstreams, block_rows):
    def _kern(x_ref, wt_ref, b_ref, *refs):
        o_ref = refs[-1]
        for s in range(nstreams):
            h = jnp.dot(refs[s][...], x_ref[...],
                        preferred_element_type=jnp.float32)
            y = jnp.dot(h, wt_ref[...],
                        preferred_element_type=jnp.float32) + b_ref[...]
            o_ref[s * block_rows:(s + 1) * block_rows, :] = jnp.maximum(y, 0.0)
    return _kern


@jax.jit
def _run(x, adj, wt, b2):
    n, d_in = x.shape
    d_out = wt.shape[1]
    s_total = NSTREAMS * BLOCK_ROWS
    assert n % s_total == 0
    grid = (n // s_total,)
    adj_specs = [
        pl.BlockSpec((BLOCK_ROWS, n),
                     lambda i, s=s: (i * NSTREAMS + s, 0))
        for s in range(NSTREAMS)
    ]
    return pl.pallas_call(
        _make_kernel(NSTREAMS, BLOCK_ROWS),
        grid=grid,
        in_specs=[
            pl.BlockSpec((n, d_in), lambda i: (0, 0)),
            pl.BlockSpec((d_in, d_out), lambda i: (0, 0)),
            pl.BlockSpec((1, d_out), lambda i: (0, 0)),
        ] + adj_specs,
        out_specs=pl.BlockSpec((s_total, d_out), lambda i: (i, 0)),
        out_shape=jax.ShapeDtypeStruct((n, d_out), jnp.float32),
        compiler_params=pltpu.CompilerParams(
            dimension_semantics=("arbitrary",),
        ),
    )(x, wt, b2, *([adj] * NSTREAMS))


def kernel(input, adj, W, b):
    wt = W.T
    b2 = b.reshape(1, -1)
    return _run(input, adj, wt, b2)


# speedup vs baseline: 1.0825x; 1.0825x over previous
import jax
import jax.numpy as jnp
from jax.experimental import pallas as pl
from jax.experimental.pallas import tpu as pltpu

NSTREAMS = 1
BLOCK_ROWS = 200


def _make_kernel(nstreams, block_rows):
    def _kern(x_ref, wt_ref, b_ref, *refs):
        o_ref = refs[-1]
        for s in range(nstreams):
            h = jnp.dot(refs[s][...], x_ref[...],
                        preferred_element_type=jnp.float32)
            y = jnp.dot(h, wt_ref[...],
                        preferred_element_type=jnp.float32) + b_ref[...]
            o_ref[s * block_rows:(s + 1) * block_rows, :] = jnp.maximum(y, 0.0)
    return _kern


@jax.jit
def _run(x, adj, wt, b2):
    n, d_in = x.shape
    d_out = wt.shape[1]
    s_total = NSTREAMS * BLOCK_ROWS
    assert n % s_total == 0
    grid = (n // s_total,)
    adj_specs = [
        pl.BlockSpec((BLOCK_ROWS, n),
                     lambda i, s=s: (i * NSTREAMS + s, 0))
        for s in range(NSTREAMS)
    ]
    return pl.pallas_call(
        _make_kernel(NSTREAMS, BLOCK_ROWS),
        grid=grid,
        in_specs=[
            pl.BlockSpec((n, d_in), lambda i: (0, 0)),
            pl.BlockSpec((d_in, d_out), lambda i: (0, 0)),
            pl.BlockSpec((1, d_out), lambda i: (0, 0)),
        ] + adj_specs,
        out_specs=pl.BlockSpec((s_total, d_out), lambda i: (i, 0)),
        out_shape=jax.ShapeDtypeStruct((n, d_out), jnp.float32),
        compiler_params=pltpu.CompilerParams(
            dimension_semantics=("arbitrary",),
        ),
    )(x, wt, b2, *([adj] * NSTREAMS))


def kernel(input, adj, W, b):
    wt = W.T
    b2 = b.reshape(1, -1)
    return _run(input, adj, wt, b2)


# speedup vs baseline: 1.1026x; 1.0186x over previous
import jax
import jax.numpy as jnp
from jax.experimental import pallas as pl
from jax.experimental.pallas import tpu as pltpu

NSTREAMS = 1
BLOCK_ROWS = 400


def _make_kernel(nstreams, block_rows):
    def _kern(x_ref, wt_ref, b_ref, *refs):
        o_ref = refs[-1]
        for s in range(nstreams):
            h = jnp.dot(refs[s][...], x_ref[...],
                        preferred_element_type=jnp.float32)
            y = jnp.dot(h, wt_ref[...],
                        preferred_element_type=jnp.float32) + b_ref[...]
            o_ref[s * block_rows:(s + 1) * block_rows, :] = jnp.maximum(y, 0.0)
    return _kern


@jax.jit
def _run(x, adj, wt, b2):
    n, d_in = x.shape
    d_out = wt.shape[1]
    s_total = NSTREAMS * BLOCK_ROWS
    assert n % s_total == 0
    grid = (n // s_total,)
    adj_specs = [
        pl.BlockSpec((BLOCK_ROWS, n),
                     lambda i, s=s: (i * NSTREAMS + s, 0))
        for s in range(NSTREAMS)
    ]
    return pl.pallas_call(
        _make_kernel(NSTREAMS, BLOCK_ROWS),
        grid=grid,
        in_specs=[
            pl.BlockSpec((n, d_in), lambda i: (0, 0)),
            pl.BlockSpec((d_in, d_out), lambda i: (0, 0)),
            pl.BlockSpec((1, d_out), lambda i: (0, 0)),
        ] + adj_specs,
        out_specs=pl.BlockSpec((s_total, d_out), lambda i: (i, 0)),
        out_shape=jax.ShapeDtypeStruct((n, d_out), jnp.float32),
        compiler_params=pltpu.CompilerParams(
            dimension_semantics=("arbitrary",),
        ),
    )(x, wt, b2, *([adj] * NSTREAMS))


def kernel(input, adj, W, b):
    wt = W.T
    b2 = b.reshape(1, -1)
    return _run(input, adj, wt, b2)
